# Initial kernel scaffold; baseline (speedup 1.0000x reference)
import jax
import jax.numpy as jnp
from jax.experimental import pallas as pl


def kernel(coords1, coords2, molecule_ix):
    raise NotImplementedError("write your pallas kernel here")


# speedup vs baseline: 15.4189x; 15.4189x over previous
import functools

import jax
import jax.numpy as jnp
from jax import lax
from jax.experimental import pallas as pl
from jax.experimental.pallas import tpu as pltpu
from jax.experimental.pallas import tpu_sc as plsc

NATOMS = 1_600_000
NMOL = 1024
NF = 18
NC = 2
NS = 16
NW = NC * NS
APW = NATOMS // NW
CHUNK = 10_000
NCHUNK = APW // CHUNK
STEPS = CHUNK // 16


def _gather16(x, idx):
    return lax.gather(
        x, idx[:, None],
        lax.GatherDimensionNumbers(
            offset_dims=(), collapsed_slice_dims=(0,), start_index_map=(0,)),
        slice_sizes=(1,),
        mode=lax.GatherScatterMode.PROMISE_IN_BOUNDS)


def _flush(table, cur, accs, newm):
    iot = lax.iota(jnp.int32, 16)
    accs = list(accs)
    for d in (1, 2, 4, 8):
        idx = jnp.minimum(iot + d, 15)
        curd = _gather16(cur, idx)
        ok = (iot < 16 - d) & (curd == cur)
        for f in range(NF):
            accs[f] = accs[f] + jnp.where(ok, _gather16(accs[f], idx), 0.0)
    prev = _gather16(cur, jnp.maximum(iot - 1, 0))
    first = ((iot == 0) | (cur != prev)) & (cur >= 0)
    for f in range(NF):
        plsc.addupdate_scatter(
            table, [jnp.full((16,), f, jnp.int32), cur], accs[f], mask=first)
    zero = jnp.zeros((16,), jnp.float32)
    return (newm,) + (zero,) * NF


def _make_seg_kernel():
    mesh = plsc.VectorSubcoreMesh(core_axis_name="c", subcore_axis_name="s")

    @functools.partial(
        pl.kernel, mesh=mesh,
        out_type=jax.ShapeDtypeStruct((NW, NF, NMOL), jnp.float32),
        compiler_params=pltpu.CompilerParams(needs_layout_passes=False),
        scratch_types=[
            pltpu.VMEM((CHUNK * 3,), jnp.float32),
            pltpu.VMEM((CHUNK * 3,), jnp.float32),
            pltpu.VMEM((CHUNK,), jnp.int32),
            pltpu.VMEM((NF, NMOL), jnp.float32),
        ],
    )
    def seg_kernel(c1_hbm, c2_hbm, mi_hbm, out_hbm, b1, b2, bm, table):
        wid = lax.axis_index("s") * NC + lax.axis_index("c")
        base = wid * APW

        def zbody(i, carry):
            z = jnp.zeros((16,), jnp.float32)
            for f in range(NF):
                table[f, pl.ds(i * 16, 16)] = z
            return carry
        lax.fori_loop(0, NMOL // 16, zbody, 0)

        iot = lax.iota(jnp.int32, 16)
        off0 = iot * 3
        off1 = off0 + 1
        off2 = off0 + 2

        def step(t, carry):
            cur = carry[0]
            m = bm[pl.ds(t * 16, 16)]
            nchanged = plsc.all_reduce_population_count(m != cur)
            anych = nchanged[0] > 0
            carry = lax.cond(
                anych,
                lambda c: _flush(table, c[0], c[1:], m),
                lambda c: c,
                carry)
            accs = list(carry[1:])
            tb = t * 48
            x1 = plsc.load_gather(b1, [tb + off0])
            y1 = plsc.load_gather(b1, [tb + off1])
            z1 = plsc.load_gather(b1, [tb + off2])
            x2 = plsc.load_gather(b2, [tb + off0])
            y2 = plsc.load_gather(b2, [tb + off1])
            z2 = plsc.load_gather(b2, [tb + off2])
            accs[0] = accs[0] + 1.0
            accs[1] = accs[1] + x1
            accs[2] = accs[2] + y1
            accs[3] = accs[3] + z1
            accs[4] = accs[4] + x2
            accs[5] = accs[5] + y2
            accs[6] = accs[6] + z2
            accs[7] = accs[7] + (x1 * x1 + y1 * y1 + z1 * z1)
            accs[8] = accs[8] + (x2 * x2 + y2 * y2 + z2 * z2)
            a1 = (x1, y1, z1)
            a2 = (x2, y2, z2)
            for j in range(3):
                for k in range(3):
                    accs[9 + 3 * j + k] = accs[9 + 3 * j + k] + a2[j] * a1[k]
            return (m,) + tuple(accs)

        def chunk_body(k, carry):
            start = base + k * CHUNK
            pltpu.sync_copy(c1_hbm.at[pl.ds(start * 3, CHUNK * 3)], b1)
            pltpu.sync_copy(c2_hbm.at[pl.ds(start * 3, CHUNK * 3)], b2)
            pltpu.sync_copy(mi_hbm.at[pl.ds(start, CHUNK)], bm)
            return lax.fori_loop(0, STEPS, step, carry)

        zero = jnp.zeros((16,), jnp.float32)
        carry = (jnp.full((16,), -1, jnp.int32),) + (zero,) * NF
        carry = lax.fori_loop(0, NCHUNK, chunk_body, carry)
        _flush(table, carry[0], carry[1:], carry[0])
        pltpu.sync_copy(table, out_hbm.at[wid])

    return seg_kernel


def _finish_body(p_ref, o_ref):
    P = p_ref[...]
    S = jnp.sum(P, axis=0)
    n = S[0]
    nc = jnp.maximum(n, 1.0)
    s1x, s1y, s1z = S[1], S[2], S[3]
    s2x, s2y, s2z = S[4], S[5], S[6]
    var1 = (S[7] - (s1x * s1x + s1y * s1y + s1z * s1z) / nc) / (3.0 * nc)
    var2 = (S[8] - (s2x * s2x + s2y * s2y + s2z * s2z) / nc) / (3.0 * nc)
    s1 = (s1x, s1y, s1z)
    s2 = (s2x, s2y, s2z)
    A = [[(S[9 + 3 * j + k] - s2[j] * s1[k] / nc) / nc for k in range(3)]
         for j in range(3)]
    detA = (A[0][0] * (A[1][1] * A[2][2] - A[1][2] * A[2][1])
            - A[0][1] * (A[1][0] * A[2][2] - A[1][2] * A[2][0])
            + A[0][2] * (A[1][0] * A[2][1] - A[1][1] * A[2][0]))
    B = [[A[0][j] * A[0][k] + A[1][j] * A[1][k] + A[2][j] * A[2][k]
          for k in range(3)] for j in range(3)]
    tr = B[0][0] + B[1][1] + B[2][2]
    mi2 = (B[0][0] * B[1][1] - B[0][1] * B[0][1]
           + B[0][0] * B[2][2] - B[0][2] * B[0][2]
           + B[1][1] * B[2][2] - B[1][2] * B[1][2])
    c0 = detA * detA
    lam = tr
    for _ in range(40):
        p = lam * lam * lam - tr * lam * lam + mi2 * lam - c0
        dp = 3.0 * lam * lam - 2.0 * tr * lam + mi2
        lam = jnp.maximum(lam - p / (dp + 1e-38), 0.0)
    s23 = tr - lam
    prod = jnp.maximum(mi2 - lam * s23, 0.0)
    disc = jnp.maximum(s23 * s23 - 4.0 * prod, 0.0)
    sq = jnp.sqrt(disc)
    l2 = jnp.maximum((s23 + sq) * 0.5, 0.0)
    l3 = jnp.maximum((s23 - sq) * 0.5, 0.0)
    sg1 = jnp.sqrt(lam)
    sg2 = jnp.sqrt(l2)
    sg3 = jnp.sqrt(l3)
    sigm = (sg1 + sg2 + jnp.where(detA < 0.0, -sg3, sg3)) / 3.0
    o_ref[...] = var1 + var2 - 2.0 * sigm


def _finish(partials):
    return pl.pallas_call(
        _finish_body,
        out_shape=jax.ShapeDtypeStruct((NMOL,), jnp.float32),
    )(partials)


@functools.cache
def _seg():
    return _make_seg_kernel()


def kernel(coords1, coords2, molecule_ix):
    partials = _seg()(coords1.reshape(-1), coords2.reshape(-1),
                      molecule_ix.astype(jnp.int32))
    return _finish(partials)


# speedup vs baseline: 484.9795x; 31.4536x over previous
import functools

import jax
import jax.numpy as jnp
from jax import lax
from jax.experimental import pallas as pl
from jax.experimental.pallas import tpu as pltpu
from jax.experimental.pallas import tpu_sc as plsc

NATOMS = 1_600_000
NMOL = 1024
NF = 18
NC = 2
NS = 16
NW = NC * NS
APW = NATOMS // NW
CHUNK = 10_000
NCHUNK = APW // CHUNK
STEPS = CHUNK // 16


def _gather16(x, idx):
    return lax.gather(
        x, idx[:, None],
        lax.GatherDimensionNumbers(
            offset_dims=(), collapsed_slice_dims=(0,), start_index_map=(0,)),
        slice_sizes=(1,),
        mode=lax.GatherScatterMode.PROMISE_IN_BOUNDS)


def _flush(table, cur, accs, newm):
    iot = lax.iota(jnp.int32, 16)
    accs = list(accs)
    for d in (1, 2, 4, 8):
        idx = jnp.minimum(iot + d, 15)
        curd = _gather16(cur, idx)
        ok = (iot < 16 - d) & (curd == cur)
        for f in range(NF):
            accs[f] = accs[f] + jnp.where(ok, _gather16(accs[f], idx), 0.0)
    prev = _gather16(cur, jnp.maximum(iot - 1, 0))
    first = ((iot == 0) | (cur != prev)) & (cur >= 0)
    for f in range(NF):
        plsc.addupdate_scatter(
            table, [jnp.full((16,), f, jnp.int32), cur], accs[f], mask=first)
    zero = jnp.zeros((16,), jnp.float32)
    return (newm,) + (zero,) * NF


def _make_seg_kernel():
    mesh = plsc.VectorSubcoreMesh(core_axis_name="c", subcore_axis_name="s")

    @functools.partial(
        pl.kernel, mesh=mesh,
        out_type=jax.ShapeDtypeStruct((NW, NF, NMOL), jnp.float32),
        compiler_params=pltpu.CompilerParams(needs_layout_passes=False),
        scratch_types=[
            pltpu.VMEM((CHUNK,), jnp.float32),
            pltpu.VMEM((CHUNK,), jnp.float32),
            pltpu.VMEM((CHUNK,), jnp.float32),
            pltpu.VMEM((CHUNK,), jnp.float32),
            pltpu.VMEM((CHUNK,), jnp.float32),
            pltpu.VMEM((CHUNK,), jnp.float32),
            pltpu.VMEM((CHUNK,), jnp.int32),
            pltpu.VMEM((NF, NMOL), jnp.float32),
        ],
    )
    def seg_kernel(x1_hbm, y1_hbm, z1_hbm, x2_hbm, y2_hbm, z2_hbm, mi_hbm,
                   out_hbm, bx1, by1, bz1, bx2, by2, bz2, bm, table):
        wid = lax.axis_index("s") * NC + lax.axis_index("c")
        base = wid * APW

        def zbody(i, carry):
            z = jnp.zeros((16,), jnp.float32)
            for f in range(NF):
                table[f, pl.ds(i * 16, 16)] = z
            return carry
        lax.fori_loop(0, NMOL // 16, zbody, 0)

        def step(t, carry):
            cur = carry[0]
            m = bm[pl.ds(t * 16, 16)]
            nchanged = plsc.all_reduce_population_count(m != cur)
            anych = nchanged[0] > 0
            carry = lax.cond(
                anych,
                lambda c: _flush(table, c[0], c[1:], m),
                lambda c: c,
                carry)
            accs = list(carry[1:])
            sl = pl.ds(t * 16, 16)
            x1 = bx1[sl]
            y1 = by1[sl]
            z1 = bz1[sl]
            x2 = bx2[sl]
            y2 = by2[sl]
            z2 = bz2[sl]
            accs[0] = accs[0] + 1.0
            accs[1] = accs[1] + x1
            accs[2] = accs[2] + y1
            accs[3] = accs[3] + z1
            accs[4] = accs[4] + x2
            accs[5] = accs[5] + y2
            accs[6] = accs[6] + z2
            accs[7] = accs[7] + (x1 * x1 + y1 * y1 + z1 * z1)
            accs[8] = accs[8] + (x2 * x2 + y2 * y2 + z2 * z2)
            a1 = (x1, y1, z1)
            a2 = (x2, y2, z2)
            for j in range(3):
                for k in range(3):
                    accs[9 + 3 * j + k] = accs[9 + 3 * j + k] + a2[j] * a1[k]
            return (m,) + tuple(accs)

        def chunk_body(k, carry):
            start = base + k * CHUNK
            sl = pl.ds(start, CHUNK)
            pltpu.sync_copy(x1_hbm.at[sl], bx1)
            pltpu.sync_copy(y1_hbm.at[sl], by1)
            pltpu.sync_copy(z1_hbm.at[sl], bz1)
            pltpu.sync_copy(x2_hbm.at[sl], bx2)
            pltpu.sync_copy(y2_hbm.at[sl], by2)
            pltpu.sync_copy(z2_hbm.at[sl], bz2)
            pltpu.sync_copy(mi_hbm.at[sl], bm)
            return lax.fori_loop(0, STEPS, step, carry)

        zero = jnp.zeros((16,), jnp.float32)
        carry = (jnp.full((16,), -1, jnp.int32),) + (zero,) * NF
        carry = lax.fori_loop(0, NCHUNK, chunk_body, carry)
        _flush(table, carry[0], carry[1:], carry[0])
        pltpu.sync_copy(table, out_hbm.at[wid])

    return seg_kernel


def _finish_body(p_ref, o_ref):
    P = p_ref[...]
    S = jnp.sum(P, axis=0)
    n = S[0]
    nc = jnp.maximum(n, 1.0)
    s1x, s1y, s1z = S[1], S[2], S[3]
    s2x, s2y, s2z = S[4], S[5], S[6]
    var1 = (S[7] - (s1x * s1x + s1y * s1y + s1z * s1z) / nc) / (3.0 * nc)
    var2 = (S[8] - (s2x * s2x + s2y * s2y + s2z * s2z) / nc) / (3.0 * nc)
    s1 = (s1x, s1y, s1z)
    s2 = (s2x, s2y, s2z)
    A = [[(S[9 + 3 * j + k] - s2[j] * s1[k] / nc) / nc for k in range(3)]
         for j in range(3)]
    detA = (A[0][0] * (A[1][1] * A[2][2] - A[1][2] * A[2][1])
            - A[0][1] * (A[1][0] * A[2][2] - A[1][2] * A[2][0])
            + A[0][2] * (A[1][0] * A[2][1] - A[1][1] * A[2][0]))
    B = [[A[0][j] * A[0][k] + A[1][j] * A[1][k] + A[2][j] * A[2][k]
          for k in range(3)] for j in range(3)]
    tr = B[0][0] + B[1][1] + B[2][2]
    mi2 = (B[0][0] * B[1][1] - B[0][1] * B[0][1]
           + B[0][0] * B[2][2] - B[0][2] * B[0][2]
           + B[1][1] * B[2][2] - B[1][2] * B[1][2])
    c0 = detA * detA
    lam = tr
    for _ in range(40):
        p = lam * lam * lam - tr * lam * lam + mi2 * lam - c0
        dp = 3.0 * lam * lam - 2.0 * tr * lam + mi2
        lam = jnp.maximum(lam - p / (dp + 1e-38), 0.0)
    s23 = tr - lam
    prod = jnp.maximum(mi2 - lam * s23, 0.0)
    disc = jnp.maximum(s23 * s23 - 4.0 * prod, 0.0)
    sq = jnp.sqrt(disc)
    l2 = jnp.maximum((s23 + sq) * 0.5, 0.0)
    l3 = jnp.maximum((s23 - sq) * 0.5, 0.0)
    sg1 = jnp.sqrt(lam)
    sg2 = jnp.sqrt(l2)
    sg3 = jnp.sqrt(l3)
    sigm = (sg1 + sg2 + jnp.where(detA < 0.0, -sg3, sg3)) / 3.0
    o_ref[...] = var1 + var2 - 2.0 * sigm


def _finish(partials):
    return pl.pallas_call(
        _finish_body,
        out_shape=jax.ShapeDtypeStruct((NMOL,), jnp.float32),
    )(partials)


@functools.cache
def _seg():
    return _make_seg_kernel()


def kernel(coords1, coords2, molecule_ix):
    partials = _seg()(coords1[:, 0], coords1[:, 1], coords1[:, 2],
                      coords2[:, 0], coords2[:, 1], coords2[:, 2],
                      molecule_ix.astype(jnp.int32))
    return _finish(partials)


# speedup vs baseline: 728.2209x; 1.5015x over previous
import functools

import jax
import jax.numpy as jnp
from jax import lax
from jax.experimental import pallas as pl
from jax.experimental.pallas import tpu as pltpu
from jax.experimental.pallas import tpu_sc as plsc

NATOMS = 1_600_000
NMOL = 1024
NF = 18
NC = 2
NS = 16
NW = NC * NS
TPW = 390
CHUNK = 1_280
NCH_MAIN = TPW * 128 // CHUNK
NCH_LAST = NCH_MAIN + 2
STEPS = CHUNK // 16


def _gather16(x, idx):
    return lax.gather(
        x, idx[:, None],
        lax.GatherDimensionNumbers(
            offset_dims=(), collapsed_slice_dims=(0,), start_index_map=(0,)),
        slice_sizes=(1,),
        mode=lax.GatherScatterMode.PROMISE_IN_BOUNDS)


def _flush(table, cur, accs, newm):
    iot = lax.iota(jnp.int32, 16)
    accs = list(accs)
    for d in (1, 2, 4, 8):
        idx = jnp.minimum(iot + d, 15)
        curd = _gather16(cur, idx)
        ok = (iot < 16 - d) & (curd == cur)
        for f in range(NF):
            accs[f] = accs[f] + jnp.where(ok, _gather16(accs[f], idx), 0.0)
    prev = _gather16(cur, jnp.maximum(iot - 1, 0))
    first = ((iot == 0) | (cur != prev)) & (cur >= 0)
    for f in range(NF):
        plsc.addupdate_scatter(
            table, [jnp.full((16,), f, jnp.int32), cur], accs[f], mask=first)
    zero = jnp.zeros((16,), jnp.float32)
    return (newm,) + (zero,) * NF


def _make_seg_kernel():
    mesh = plsc.VectorSubcoreMesh(core_axis_name="c", subcore_axis_name="s")

    @functools.partial(
        pl.kernel, mesh=mesh,
        out_type=jax.ShapeDtypeStruct((NW, NF, NMOL), jnp.float32),
        compiler_params=pltpu.CompilerParams(needs_layout_passes=False),
        scratch_types=[
            pltpu.VMEM((3, CHUNK), jnp.float32),
            pltpu.VMEM((3, CHUNK), jnp.float32),
            pltpu.VMEM((CHUNK,), jnp.int32),
            pltpu.VMEM((NF, NMOL), jnp.float32),
        ],
    )
    def seg_kernel(ct1_hbm, ct2_hbm, mi_hbm, out_hbm, b1, b2, bm, table):
        wid = lax.axis_index("s") * NC + lax.axis_index("c")
        base = wid * (TPW * 128)
        nch = jnp.where(wid == NW - 1, NCH_LAST, NCH_MAIN)

        def zbody(i, carry):
            z = jnp.zeros((16,), jnp.float32)
            for f in range(NF):
                table[f, pl.ds(i * 16, 16)] = z
            return carry
        lax.fori_loop(0, NMOL // 16, zbody, 0)

        def step(t, carry):
            cur = carry[0]
            m = bm[pl.ds(t * 16, 16)]
            nchanged = plsc.all_reduce_population_count(m != cur)
            anych = nchanged[0] > 0
            carry = lax.cond(
                anych,
                lambda c: _flush(table, c[0], c[1:], m),
                lambda c: c,
                carry)
            accs = list(carry[1:])
            sl = pl.ds(t * 16, 16)
            x1 = b1[0, sl]
            y1 = b1[1, sl]
            z1 = b1[2, sl]
            x2 = b2[0, sl]
            y2 = b2[1, sl]
            z2 = b2[2, sl]
            accs[0] = accs[0] + 1.0
            accs[1] = accs[1] + x1
            accs[2] = accs[2] + y1
            accs[3] = accs[3] + z1
            accs[4] = accs[4] + x2
            accs[5] = accs[5] + y2
            accs[6] = accs[6] + z2
            accs[7] = accs[7] + (x1 * x1 + y1 * y1 + z1 * z1)
            accs[8] = accs[8] + (x2 * x2 + y2 * y2 + z2 * z2)
            a1 = (x1, y1, z1)
            a2 = (x2, y2, z2)
            for j in range(3):
                for k in range(3):
                    accs[9 + 3 * j + k] = accs[9 + 3 * j + k] + a2[j] * a1[k]
            return (m,) + tuple(accs)

        def chunk_body(k, carry):
            start = base + k * CHUNK
            sl = pl.ds(start, CHUNK)
            pltpu.sync_copy(ct1_hbm.at[:, sl], b1)
            pltpu.sync_copy(ct2_hbm.at[:, sl], b2)
            pltpu.sync_copy(mi_hbm.at[sl], bm)
            return lax.fori_loop(0, STEPS, step, carry)

        zero = jnp.zeros((16,), jnp.float32)
        carry = (jnp.full((16,), -1, jnp.int32),) + (zero,) * NF
        carry = lax.fori_loop(0, nch, chunk_body, carry)
        _flush(table, carry[0], carry[1:], carry[0])
        pltpu.sync_copy(table, out_hbm.at[wid])

    return seg_kernel


def _finish_body(p_ref, o_ref):
    P = p_ref[...]
    S = jnp.sum(P, axis=0)
    n = S[0]
    nc = jnp.maximum(n, 1.0)
    s1x, s1y, s1z = S[1], S[2], S[3]
    s2x, s2y, s2z = S[4], S[5], S[6]
    var1 = (S[7] - (s1x * s1x + s1y * s1y + s1z * s1z) / nc) / (3.0 * nc)
    var2 = (S[8] - (s2x * s2x + s2y * s2y + s2z * s2z) / nc) / (3.0 * nc)
    s1 = (s1x, s1y, s1z)
    s2 = (s2x, s2y, s2z)
    A = [[(S[9 + 3 * j + k] - s2[j] * s1[k] / nc) / nc for k in range(3)]
         for j in range(3)]
    detA = (A[0][0] * (A[1][1] * A[2][2] - A[1][2] * A[2][1])
            - A[0][1] * (A[1][0] * A[2][2] - A[1][2] * A[2][0])
            + A[0][2] * (A[1][0] * A[2][1] - A[1][1] * A[2][0]))
    B = [[A[0][j] * A[0][k] + A[1][j] * A[1][k] + A[2][j] * A[2][k]
          for k in range(3)] for j in range(3)]
    tr = B[0][0] + B[1][1] + B[2][2]
    mi2 = (B[0][0] * B[1][1] - B[0][1] * B[0][1]
           + B[0][0] * B[2][2] - B[0][2] * B[0][2]
           + B[1][1] * B[2][2] - B[1][2] * B[1][2])
    c0 = detA * detA
    lam = tr
    for _ in range(40):
        p = lam * lam * lam - tr * lam * lam + mi2 * lam - c0
        dp = 3.0 * lam * lam - 2.0 * tr * lam + mi2
        lam = jnp.maximum(lam - p / (dp + 1e-38), 0.0)
    s23 = tr - lam
    prod = jnp.maximum(mi2 - lam * s23, 0.0)
    disc = jnp.maximum(s23 * s23 - 4.0 * prod, 0.0)
    sq = jnp.sqrt(disc)
    l2 = jnp.maximum((s23 + sq) * 0.5, 0.0)
    l3 = jnp.maximum((s23 - sq) * 0.5, 0.0)
    sg1 = jnp.sqrt(lam)
    sg2 = jnp.sqrt(l2)
    sg3 = jnp.sqrt(l3)
    sigm = (sg1 + sg2 + jnp.where(detA < 0.0, -sg3, sg3)) / 3.0
    o_ref[...] = var1 + var2 - 2.0 * sigm


def _finish(partials):
    return pl.pallas_call(
        _finish_body,
        out_shape=jax.ShapeDtypeStruct((NMOL,), jnp.float32),
    )(partials)


@functools.cache
def _seg():
    return _make_seg_kernel()


def kernel(coords1, coords2, molecule_ix):
    partials = _seg()(coords1.T, coords2.T, molecule_ix.astype(jnp.int32))
    return _finish(partials)


# speedup vs baseline: 1881.9454x; 2.5843x over previous
import functools

import jax
import jax.numpy as jnp
from jax import lax
from jax.experimental import pallas as pl
from jax.experimental.pallas import tpu as pltpu
from jax.experimental.pallas import tpu_sc as plsc

NATOMS = 1_600_000
NMOL = 1024
NF = 18
NC = 2
NS = 16
NW = NC * NS
TPW = 400
CHUNK = 1_280
NCH_MAIN = TPW * 128 // CHUNK
NCH_LAST = (12_500 - 31 * TPW) * 128 // CHUNK
STEPS = CHUNK // 16
G = 4
NGROUP = STEPS // G


def _gather16(x, idx):
    return lax.gather(
        x, idx[:, None],
        lax.GatherDimensionNumbers(
            offset_dims=(), collapsed_slice_dims=(0,), start_index_map=(0,)),
        slice_sizes=(1,),
        mode=lax.GatherScatterMode.PROMISE_IN_BOUNDS)


def _flush(table, cur, accs, newm):
    iot = lax.iota(jnp.int32, 16)
    accs = list(accs)
    for d in (1, 2, 4, 8):
        idx = jnp.minimum(iot + d, 15)
        curd = _gather16(cur, idx)
        ok = (iot < 16 - d) & (curd == cur)
        for f in range(NF):
            accs[f] = accs[f] + jnp.where(ok, _gather16(accs[f], idx), 0.0)
    prev = _gather16(cur, jnp.maximum(iot - 1, 0))
    first = ((iot == 0) | (cur != prev)) & (cur >= 0)
    for f in range(NF):
        plsc.addupdate_scatter(
            table, [jnp.full((16,), f, jnp.int32), cur], accs[f], mask=first)
    zero = jnp.zeros((16,), jnp.float32)
    return (newm,) + (zero,) * NF


def _make_seg_kernel():
    mesh = plsc.VectorSubcoreMesh(core_axis_name="c", subcore_axis_name="s")

    @functools.partial(
        pl.kernel, mesh=mesh,
        out_type=jax.ShapeDtypeStruct((NW, NF, NMOL), jnp.float32),
        compiler_params=pltpu.CompilerParams(needs_layout_passes=False),
        scratch_types=[
            pltpu.VMEM((3, CHUNK), jnp.float32),
            pltpu.VMEM((3, CHUNK), jnp.float32),
            pltpu.VMEM((CHUNK,), jnp.int32),
            pltpu.VMEM((3, CHUNK), jnp.float32),
            pltpu.VMEM((3, CHUNK), jnp.float32),
            pltpu.VMEM((CHUNK,), jnp.int32),
            pltpu.VMEM((NF, NMOL), jnp.float32),
            pltpu.SemaphoreType.DMA,
            pltpu.SemaphoreType.DMA,
        ],
    )
    def seg_kernel(ct1_hbm, ct2_hbm, mi_hbm, out_hbm,
                   b1a, b2a, bma, b1b, b2b, bmb, table, sema, semb):
        wid = lax.axis_index("s") * NC + lax.axis_index("c")
        base = wid * (TPW * 128)
        nch = jnp.where(wid == NW - 1, NCH_LAST, NCH_MAIN)
        bufs_a = (b1a, b2a, bma)
        bufs_b = (b1b, b2b, bmb)

        def zbody(i, carry):
            z = jnp.zeros((16,), jnp.float32)
            for f in range(NF):
                table[f, pl.ds(i * 16, 16)] = z
            return carry
        lax.fori_loop(0, NMOL // 16, zbody, 0)

        def issue(c, bufs, sem):
            start = base + c * CHUNK
            sl = pl.ds(start, CHUNK)
            pltpu.async_copy(ct1_hbm.at[:, sl], bufs[0], sem)
            pltpu.async_copy(ct2_hbm.at[:, sl], bufs[1], sem)
            pltpu.async_copy(mi_hbm.at[sl], bufs[2], sem)

        def wait(bufs, sem):
            sl = pl.ds(0, CHUNK)
            pltpu.make_async_copy(ct1_hbm.at[:, sl], bufs[0], sem).wait()
            pltpu.make_async_copy(ct2_hbm.at[:, sl], bufs[1], sem).wait()
            pltpu.make_async_copy(mi_hbm.at[sl], bufs[2], sem).wait()

        def accumulate(accs, b1, b2, sl):
            x1 = b1[0, sl]
            y1 = b1[1, sl]
            z1 = b1[2, sl]
            x2 = b2[0, sl]
            y2 = b2[1, sl]
            z2 = b2[2, sl]
            accs[0] = accs[0] + 1.0
            accs[1] = accs[1] + x1
            accs[2] = accs[2] + y1
            accs[3] = accs[3] + z1
            accs[4] = accs[4] + x2
            accs[5] = accs[5] + y2
            accs[6] = accs[6] + z2
            accs[7] = accs[7] + (x1 * x1 + y1 * y1 + z1 * z1)
            accs[8] = accs[8] + (x2 * x2 + y2 * y2 + z2 * z2)
            a1 = (x1, y1, z1)
            a2 = (x2, y2, z2)
            for j in range(3):
                for k in range(3):
                    accs[9 + 3 * j + k] = accs[9 + 3 * j + k] + a2[j] * a1[k]
            return accs

        def process(bufs, carry):
            b1, b2, bm = bufs

            def window(t, c):
                cur = c[0]
                m = bm[pl.ds(t * 16, 16)]
                nchanged = plsc.all_reduce_population_count(m != cur)
                c = lax.cond(
                    nchanged[0] > 0,
                    lambda c: _flush(table, c[0], c[1:], m),
                    lambda c: c,
                    c)
                return (c[0],) + tuple(
                    accumulate(list(c[1:]), b1, b2, pl.ds(t * 16, 16)))

            def group(g, c):
                cur = c[0]
                tb = g * G
                m_first = bm[pl.ds(tb * 16, 16)]
                m_last = bm[pl.ds((tb + G - 1) * 16, 16)]
                diff = (m_first != cur) | (m_last != m_first)
                nd = plsc.all_reduce_population_count(diff)

                def fast_fn(c):
                    accs = list(c[1:])
                    for j in range(G):
                        accs = accumulate(accs, b1, b2,
                                          pl.ds((tb + j) * 16, 16))
                    return (c[0],) + tuple(accs)

                def slow_fn(c):
                    return lax.fori_loop(tb, tb + G, window, c)

                return lax.cond(nd[0] == 0, fast_fn, slow_fn, c)

            return lax.fori_loop(0, NGROUP, group, carry)

        zero = jnp.zeros((16,), jnp.float32)
        carry = (jnp.full((16,), -1, jnp.int32),) + (zero,) * NF
        issue(0, bufs_a, sema)

        def pair(k, carry):
            issue(2 * k + 1, bufs_b, semb)
            wait(bufs_a, sema)
            carry = process(bufs_a, carry)
            pl.when(2 * k + 2 < nch)(lambda: issue(2 * k + 2, bufs_a, sema))
            wait(bufs_b, semb)
            return process(bufs_b, carry)

        carry = lax.fori_loop(0, nch // 2, pair, carry)
        _flush(table, carry[0], carry[1:], carry[0])
        pltpu.sync_copy(table, out_hbm.at[wid])

    return seg_kernel


def _finish_body(p_ref, o_ref):
    P = p_ref[...]
    S = jnp.sum(P, axis=0)
    n = S[0]
    nc = jnp.maximum(n, 1.0)
    s1x, s1y, s1z = S[1], S[2], S[3]
    s2x, s2y, s2z = S[4], S[5], S[6]
    var1 = (S[7] - (s1x * s1x + s1y * s1y + s1z * s1z) / nc) / (3.0 * nc)
    var2 = (S[8] - (s2x * s2x + s2y * s2y + s2z * s2z) / nc) / (3.0 * nc)
    s1 = (s1x, s1y, s1z)
    s2 = (s2x, s2y, s2z)
    A = [[(S[9 + 3 * j + k] - s2[j] * s1[k] / nc) / nc for k in range(3)]
         for j in range(3)]
    detA = (A[0][0] * (A[1][1] * A[2][2] - A[1][2] * A[2][1])
            - A[0][1] * (A[1][0] * A[2][2] - A[1][2] * A[2][0])
            + A[0][2] * (A[1][0] * A[2][1] - A[1][1] * A[2][0]))
    B = [[A[0][j] * A[0][k] + A[1][j] * A[1][k] + A[2][j] * A[2][k]
          for k in range(3)] for j in range(3)]
    tr = B[0][0] + B[1][1] + B[2][2]
    mi2 = (B[0][0] * B[1][1] - B[0][1] * B[0][1]
           + B[0][0] * B[2][2] - B[0][2] * B[0][2]
           + B[1][1] * B[2][2] - B[1][2] * B[1][2])
    c0 = detA * detA
    lam = tr
    for _ in range(40):
        p = lam * lam * lam - tr * lam * lam + mi2 * lam - c0
        dp = 3.0 * lam * lam - 2.0 * tr * lam + mi2
        lam = jnp.maximum(lam - p / (dp + 1e-38), 0.0)
    s23 = tr - lam
    prod = jnp.maximum(mi2 - lam * s23, 0.0)
    disc = jnp.maximum(s23 * s23 - 4.0 * prod, 0.0)
    sq = jnp.sqrt(disc)
    l2 = jnp.maximum((s23 + sq) * 0.5, 0.0)
    l3 = jnp.maximum((s23 - sq) * 0.5, 0.0)
    sg1 = jnp.sqrt(lam)
    sg2 = jnp.sqrt(l2)
    sg3 = jnp.sqrt(l3)
    sigm = (sg1 + sg2 + jnp.where(detA < 0.0, -sg3, sg3)) / 3.0
    o_ref[...] = var1 + var2 - 2.0 * sigm


def _finish(partials):
    return pl.pallas_call(
        _finish_body,
        out_shape=jax.ShapeDtypeStruct((NMOL,), jnp.float32),
    )(partials)


@functools.cache
def _seg():
    return _make_seg_kernel()


def kernel(coords1, coords2, molecule_ix):
    partials = _seg()(coords1.T, coords2.T, molecule_ix.astype(jnp.int32))
    return _finish(partials)


# speedup vs baseline: 2054.3187x; 1.0916x over previous
import functools

import jax
import jax.numpy as jnp
from jax import lax
from jax.experimental import pallas as pl
from jax.experimental.pallas import tpu as pltpu
from jax.experimental.pallas import tpu_sc as plsc

NATOMS = 1_600_000
NMOL = 1024
NF = 18
NC = 2
NS = 16
NW = NC * NS
TPW = 400
CHUNK = 1_280
NCH_MAIN = TPW * 128 // CHUNK
NCH_LAST = (12_500 - 31 * TPW) * 128 // CHUNK
STEPS = CHUNK // 16
G = 8
NGROUP = STEPS // G


def _gather16(x, idx):
    return lax.gather(
        x, idx[:, None],
        lax.GatherDimensionNumbers(
            offset_dims=(), collapsed_slice_dims=(0,), start_index_map=(0,)),
        slice_sizes=(1,),
        mode=lax.GatherScatterMode.PROMISE_IN_BOUNDS)


def _flush(table, cur, accs, newm):
    iot = lax.iota(jnp.int32, 16)
    accs = list(accs)
    for d in (1, 2, 4, 8):
        idx = jnp.minimum(iot + d, 15)
        curd = _gather16(cur, idx)
        ok = (iot < 16 - d) & (curd == cur)
        for f in range(NF):
            accs[f] = accs[f] + jnp.where(ok, _gather16(accs[f], idx), 0.0)
    prev = _gather16(cur, jnp.maximum(iot - 1, 0))
    first = ((iot == 0) | (cur != prev)) & (cur >= 0)
    for f in range(NF):
        plsc.addupdate_scatter(
            table, [jnp.full((16,), f, jnp.int32), cur], accs[f], mask=first)
    zero = jnp.zeros((16,), jnp.float32)
    return (newm,) + (zero,) * NF


def _make_seg_kernel():
    mesh = plsc.VectorSubcoreMesh(core_axis_name="c", subcore_axis_name="s")

    @functools.partial(
        pl.kernel, mesh=mesh,
        out_type=jax.ShapeDtypeStruct((NW, NF, NMOL), jnp.float32),
        compiler_params=pltpu.CompilerParams(needs_layout_passes=False),
        scratch_types=[
            pltpu.VMEM((3, CHUNK), jnp.float32),
            pltpu.VMEM((3, CHUNK), jnp.float32),
            pltpu.VMEM((CHUNK,), jnp.int32),
            pltpu.VMEM((3, CHUNK), jnp.float32),
            pltpu.VMEM((3, CHUNK), jnp.float32),
            pltpu.VMEM((CHUNK,), jnp.int32),
            pltpu.VMEM((NF, NMOL), jnp.float32),
            pltpu.SemaphoreType.DMA,
            pltpu.SemaphoreType.DMA,
        ],
    )
    def seg_kernel(ct1_hbm, ct2_hbm, mi_hbm, out_hbm,
                   b1a, b2a, bma, b1b, b2b, bmb, table, sema, semb):
        wid = lax.axis_index("s") * NC + lax.axis_index("c")
        base = wid * (TPW * 128)
        nch = jnp.where(wid == NW - 1, NCH_LAST, NCH_MAIN)
        bufs_a = (b1a, b2a, bma)
        bufs_b = (b1b, b2b, bmb)

        def zbody(i, carry):
            z = jnp.zeros((16,), jnp.float32)
            for f in range(NF):
                table[f, pl.ds(i * 16, 16)] = z
            return carry
        lax.fori_loop(0, NMOL // 16, zbody, 0)

        def issue(c, bufs, sem):
            start = base + c * CHUNK
            sl = pl.ds(start, CHUNK)
            pltpu.async_copy(ct1_hbm.at[:, sl], bufs[0], sem)
            pltpu.async_copy(ct2_hbm.at[:, sl], bufs[1], sem)
            pltpu.async_copy(mi_hbm.at[sl], bufs[2], sem)

        def wait(bufs, sem):
            sl = pl.ds(0, CHUNK)
            pltpu.make_async_copy(ct1_hbm.at[:, sl], bufs[0], sem).wait()
            pltpu.make_async_copy(ct2_hbm.at[:, sl], bufs[1], sem).wait()
            pltpu.make_async_copy(mi_hbm.at[sl], bufs[2], sem).wait()

        def accumulate(accs, b1, b2, sl):
            x1 = b1[0, sl]
            y1 = b1[1, sl]
            z1 = b1[2, sl]
            x2 = b2[0, sl]
            y2 = b2[1, sl]
            z2 = b2[2, sl]
            accs[0] = accs[0] + 1.0
            accs[1] = accs[1] + x1
            accs[2] = accs[2] + y1
            accs[3] = accs[3] + z1
            accs[4] = accs[4] + x2
            accs[5] = accs[5] + y2
            accs[6] = accs[6] + z2
            accs[7] = accs[7] + (x1 * x1 + y1 * y1 + z1 * z1)
            accs[8] = accs[8] + (x2 * x2 + y2 * y2 + z2 * z2)
            a1 = (x1, y1, z1)
            a2 = (x2, y2, z2)
            for j in range(3):
                for k in range(3):
                    accs[9 + 3 * j + k] = accs[9 + 3 * j + k] + a2[j] * a1[k]
            return accs

        def process(bufs, carry):
            b1, b2, bm = bufs

            def window(t, c):
                cur = c[0]
                m = bm[pl.ds(t * 16, 16)]
                nchanged = plsc.all_reduce_population_count(m != cur)
                c = lax.cond(
                    nchanged[0] > 0,
                    lambda c: _flush(table, c[0], c[1:], m),
                    lambda c: c,
                    c)
                return (c[0],) + tuple(
                    accumulate(list(c[1:]), b1, b2, pl.ds(t * 16, 16)))

            def group(g, c):
                cur = c[0]
                tb = g * G
                m_first = bm[pl.ds(tb * 16, 16)]
                m_last = bm[pl.ds((tb + G - 1) * 16, 16)]
                diff = (m_first != cur) | (m_last != m_first)
                nd = plsc.all_reduce_population_count(diff)

                def fast_fn(c):
                    accs = list(c[1:])
                    for j in range(G):
                        accs = accumulate(accs, b1, b2,
                                          pl.ds((tb + j) * 16, 16))
                    return (c[0],) + tuple(accs)

                def slow_fn(c):
                    return lax.fori_loop(tb, tb + G, window, c)

                return lax.cond(nd[0] == 0, fast_fn, slow_fn, c)

            return lax.fori_loop(0, NGROUP, group, carry)

        zero = jnp.zeros((16,), jnp.float32)
        carry = (jnp.full((16,), -1, jnp.int32),) + (zero,) * NF
        issue(0, bufs_a, sema)

        def pair(k, carry):
            issue(2 * k + 1, bufs_b, semb)
            wait(bufs_a, sema)
            carry = process(bufs_a, carry)
            pl.when(2 * k + 2 < nch)(lambda: issue(2 * k + 2, bufs_a, sema))
            wait(bufs_b, semb)
            return process(bufs_b, carry)

        carry = lax.fori_loop(0, nch // 2, pair, carry)
        _flush(table, carry[0], carry[1:], carry[0])
        pltpu.sync_copy(table, out_hbm.at[wid])

    return seg_kernel


def _finish_body(p_ref, o_ref):
    P = p_ref[...]
    S = jnp.sum(P, axis=0)
    n = S[0]
    nc = jnp.maximum(n, 1.0)
    s1x, s1y, s1z = S[1], S[2], S[3]
    s2x, s2y, s2z = S[4], S[5], S[6]
    var1 = (S[7] - (s1x * s1x + s1y * s1y + s1z * s1z) / nc) / (3.0 * nc)
    var2 = (S[8] - (s2x * s2x + s2y * s2y + s2z * s2z) / nc) / (3.0 * nc)
    s1 = (s1x, s1y, s1z)
    s2 = (s2x, s2y, s2z)
    A = [[(S[9 + 3 * j + k] - s2[j] * s1[k] / nc) / nc for k in range(3)]
         for j in range(3)]
    detA = (A[0][0] * (A[1][1] * A[2][2] - A[1][2] * A[2][1])
            - A[0][1] * (A[1][0] * A[2][2] - A[1][2] * A[2][0])
            + A[0][2] * (A[1][0] * A[2][1] - A[1][1] * A[2][0]))
    B = [[A[0][j] * A[0][k] + A[1][j] * A[1][k] + A[2][j] * A[2][k]
          for k in range(3)] for j in range(3)]
    tr = B[0][0] + B[1][1] + B[2][2]
    mi2 = (B[0][0] * B[1][1] - B[0][1] * B[0][1]
           + B[0][0] * B[2][2] - B[0][2] * B[0][2]
           + B[1][1] * B[2][2] - B[1][2] * B[1][2])
    c0 = detA * detA
    lam = tr
    for _ in range(40):
        p = lam * lam * lam - tr * lam * lam + mi2 * lam - c0
        dp = 3.0 * lam * lam - 2.0 * tr * lam + mi2
        lam = jnp.maximum(lam - p / (dp + 1e-38), 0.0)
    s23 = tr - lam
    prod = jnp.maximum(mi2 - lam * s23, 0.0)
    disc = jnp.maximum(s23 * s23 - 4.0 * prod, 0.0)
    sq = jnp.sqrt(disc)
    l2 = jnp.maximum((s23 + sq) * 0.5, 0.0)
    l3 = jnp.maximum((s23 - sq) * 0.5, 0.0)
    sg1 = jnp.sqrt(lam)
    sg2 = jnp.sqrt(l2)
    sg3 = jnp.sqrt(l3)
    sigm = (sg1 + sg2 + jnp.where(detA < 0.0, -sg3, sg3)) / 3.0
    o_ref[...] = var1 + var2 - 2.0 * sigm


def _finish(partials):
    return pl.pallas_call(
        _finish_body,
        out_shape=jax.ShapeDtypeStruct((NMOL,), jnp.float32),
    )(partials)


@functools.cache
def _seg():
    return _make_seg_kernel()


def kernel(coords1, coords2, molecule_ix):
    partials = _seg()(coords1.T, coords2.T, molecule_ix.astype(jnp.int32))
    return _finish(partials)
